# Initial kernel scaffold; baseline (speedup 1.0000x reference)
import jax
import jax.numpy as jnp
from jax.experimental import pallas as pl


def kernel(h, edge_index, W1, b1, W2, b2):
    raise NotImplementedError("write your pallas kernel here")


# speedup vs baseline: 28.3845x; 28.3845x over previous
import functools

import jax
import jax.numpy as jnp
from jax import lax
from jax.experimental import pallas as pl
from jax.experimental.pallas import tpu as pltpu
from jax.experimental.pallas import tpu_sc as plsc

N = 10000
E = 320000
D = 128

NC = 2
NS = 16
NW = NC * NS
CH = 128
NCH = 80
G = 16
NG = NCH // G
EPAD = NW * NCH * CH
NPAD = 10240
RPT = NPAD // NS
BLK = 1024
GRID = NPAD // BLK


def _fill_f32(ref, n, value):
    v = jnp.full((16,), value, jnp.float32)

    def body(i, c):
        ref[pl.ds(i * 16, 16)] = v
        return c

    lax.fori_loop(0, n // 16, body, 0)


def _fill_rows_zero(ref, nrows):
    z = jnp.zeros((16,), jnp.float32)

    def body(r, c):
        for k in range(8):
            ref[r, pl.ds(k * 16, 16)] = z
        return c

    lax.fori_loop(0, nrows, body, 0)


def _deg_body(idx_hbm, deg_out, idx_v, ones_v, zer_v, deg_s):
    c = lax.axis_index("c")
    s = lax.axis_index("s")
    wid = c * NS + s

    pltpu.sync_copy(idx_hbm.at[wid], idx_v)
    _fill_f32(ones_v, CH, 1.0)
    _fill_f32(zer_v, RPT, 0.0)
    pltpu.sync_copy(zer_v, deg_s.at[pl.ds(s * RPT, RPT)])
    plsc.subcore_barrier()
    for j in range(NCH):
        pltpu.sync_copy(ones_v, deg_s.at[idx_v.at[j, 0]], add=True)
    plsc.subcore_barrier()
    pltpu.sync_copy(deg_s.at[pl.ds(s * RPT, RPT)],
                    deg_out.at[c, pl.ds(s * RPT, RPT)])


_sc_deg = pl.kernel(
    _deg_body,
    out_type=jax.ShapeDtypeStruct((NC, NPAD), jnp.float32),
    mesh=plsc.VectorSubcoreMesh(
        core_axis_name="c", subcore_axis_name="s", num_cores=NC,
        num_subcores=NS),
    scratch_types=[
        pltpu.VMEM((NCH, 2, CH), jnp.int32),
        pltpu.VMEM((CH,), jnp.float32),
        pltpu.VMEM((RPT,), jnp.float32),
        pltpu.VMEM_SHARED((NPAD,), jnp.float32),
    ],
)


def _agg_body(y_hbm, idx_hbm, out_hbm,
              ib0, ib1, buf0, buf1, acc_s, isem0, isem1, sem0, sem1):
    c = lax.axis_index("c")
    s = lax.axis_index("s")
    wid = c * NS + s

    ibs = (ib0, ib1)
    isems = (isem0, isem1)
    ih = [None, None]
    ih[0] = pltpu.async_copy(idx_hbm.at[wid, pl.ds(0, G)], ib0, isem0)
    if NG > 1:
        ih[1] = pltpu.async_copy(idx_hbm.at[wid, pl.ds(G, G)], ib1, isem1)

    _fill_rows_zero(buf0, CH)
    for t in range(RPT // CH):
        pltpu.sync_copy(buf0, acc_s.at[pl.ds(s * RPT + t * CH, CH)])
    plsc.subcore_barrier()

    bufs = (buf0, buf1)
    sems = (sem0, sem1)
    rh = [None, None]
    ih[0].wait()
    rh[0] = pltpu.async_copy(y_hbm.at[ib0.at[0, 0]], buf0, sem0)
    for j in range(NCH):
        g, k = divmod(j, G)
        cb = j & 1
        nb = cb ^ 1
        if j + 1 < NCH:
            ng, nk = divmod(j + 1, G)
            if nk == 0:
                ih[ng % 2].wait()
            rh[nb] = pltpu.async_copy(
                y_hbm.at[ibs[ng % 2].at[nk, 0]], bufs[nb], sems[nb])
        rh[cb].wait()
        pltpu.sync_copy(bufs[cb], acc_s.at[ibs[g % 2].at[k, 1]], add=True)
        if k == G - 1 and g + 2 < NG:
            ih[g % 2] = pltpu.async_copy(
                idx_hbm.at[wid, pl.ds((g + 2) * G, G)], ibs[g % 2],
                isems[g % 2])
    plsc.subcore_barrier()

    for t in range(RPT // CH):
        base = s * RPT + t * CH
        pltpu.sync_copy(acc_s.at[pl.ds(base, CH)],
                        out_hbm.at[c, pl.ds(base, CH)])


_sc_agg = pl.kernel(
    _agg_body,
    out_type=jax.ShapeDtypeStruct((NC, NPAD, D), jnp.float32),
    mesh=plsc.VectorSubcoreMesh(
        core_axis_name="c", subcore_axis_name="s", num_cores=NC,
        num_subcores=NS),
    scratch_types=[
        pltpu.VMEM((G, 2, CH), jnp.int32),
        pltpu.VMEM((G, 2, CH), jnp.int32),
        pltpu.VMEM((CH, D), jnp.float32),
        pltpu.VMEM((CH, D), jnp.float32),
        pltpu.VMEM_SHARED((NPAD, D), jnp.float32),
        pltpu.SemaphoreType.DMA,
        pltpu.SemaphoreType.DMA,
        pltpu.SemaphoreType.DMA,
        pltpu.SemaphoreType.DMA,
    ],
)


def _s2d(dp):
    deg = dp[0:1, :] + dp[1:2, :]
    sc = lax.rsqrt(jnp.maximum(deg, 1.0))
    ones = jnp.ones((1, D), jnp.float32)
    return lax.dot_general(sc, ones, (((0,), (0,)), ((), ())),
                           preferred_element_type=jnp.float32)


def _k1_body(dp_ref, h_ref, w_ref, y_ref):
    s2d = _s2d(dp_ref[...])
    y_ref[...] = jnp.dot(h_ref[...] * s2d, w_ref[...],
                         preferred_element_type=jnp.float32)


def _k3_body(dp_ref, p_ref, b_ref, w_ref, y_ref):
    s2d = _s2d(dp_ref[...])
    agg = (p_ref[0] + p_ref[1]) * s2d + jnp.reshape(b_ref[...], (1, D))
    x1 = jnp.maximum(agg, 0.0)
    y_ref[...] = jnp.dot(x1 * s2d, w_ref[...],
                         preferred_element_type=jnp.float32)


def _k5_body(dp_ref, q_ref, b_ref, o_ref):
    s2d = _s2d(dp_ref[...])
    o_ref[...] = (q_ref[0] + q_ref[1]) * s2d + jnp.reshape(b_ref[...], (1, D))


_dp_spec = pl.BlockSpec((NC, BLK), lambda i: (0, i))
_row_spec = pl.BlockSpec((BLK, D), lambda i: (i, 0))
_mat_spec = pl.BlockSpec((D, D), lambda i: (0, 0))
_vec_spec = pl.BlockSpec((D,), lambda i: (0,))
_par_spec = pl.BlockSpec((NC, BLK, D), lambda i: (0, i, 0))
_out_struct = jax.ShapeDtypeStruct((NPAD, D), jnp.float32)

_tc_k1 = pl.pallas_call(
    _k1_body, grid=(GRID,),
    in_specs=[_dp_spec, _row_spec, _mat_spec],
    out_specs=_row_spec, out_shape=_out_struct)

_tc_k3 = pl.pallas_call(
    _k3_body, grid=(GRID,),
    in_specs=[_dp_spec, _par_spec, _vec_spec, _mat_spec],
    out_specs=_row_spec, out_shape=_out_struct)

_tc_k5 = pl.pallas_call(
    _k5_body, grid=(GRID,),
    in_specs=[_dp_spec, _par_spec, _vec_spec],
    out_specs=_row_spec, out_shape=_out_struct)


def kernel(h, edge_index, W1, b1, W2, b2):
    src = edge_index[0].astype(jnp.int32)
    dst = edge_index[1].astype(jnp.int32)
    pad = EPAD - E
    padv = N + (jnp.arange(pad, dtype=jnp.int32) % (NPAD - N))
    src_p = jnp.concatenate([src, padv]).reshape(NW, NCH, 1, CH)
    dst_p = jnp.concatenate([dst, padv]).reshape(NW, NCH, 1, CH)
    idx_p = jnp.concatenate([src_p, dst_p], axis=2)
    h_pad = jnp.zeros((NPAD, D), jnp.float32).at[:N].set(h)

    deg_part = _sc_deg(idx_p)
    y1 = _tc_k1(deg_part, h_pad, W1)
    p = _sc_agg(y1, idx_p)
    y2 = _tc_k3(deg_part, p, b1, W2)
    q = _sc_agg(y2, idx_p)
    out = _tc_k5(deg_part, q, b2)
    return out[:N]
